# baseline (speedup 1.0000x reference)
import jax
import jax.numpy as jnp
from jax.experimental import pallas as pl

_ROWS_PER_BLOCK = 16


def _upsample_block(x_ref, o_ref):
    x = x_ref[0]
    ib, w, c = x.shape
    y = jnp.broadcast_to(x[:, None, :, None, :], (ib, 2, w, 2, c))
    o_ref[0] = y.reshape(2 * ib, 2 * w, c)


def kernel(img):
    n, h, w, c = img.shape
    ib = _ROWS_PER_BLOCK
    return pl.pallas_call(
        _upsample_block,
        grid=(n, h // ib),
        in_specs=[pl.BlockSpec((1, ib, w, c), lambda b, i: (b, i, 0, 0))],
        out_specs=pl.BlockSpec((1, 2 * ib, 2 * w, c), lambda b, i: (b, i, 0, 0)),
        out_shape=jax.ShapeDtypeStruct((n, 2 * h, 2 * w, c), img.dtype),
    )(img)
